# baseline (speedup 1.0000x reference)
import dataclasses
import functools

import jax
import jax.numpy as jnp
from jax.experimental import pallas as pl
from jax.experimental.pallas import tpu as pltpu
from jax.experimental.pallas import tpu_sc as plsc

_K = 32
_K_PAD = 33
_SC_CH = 128
_SC_TT = 64


def _subtree(xb, t, yv, k0, k1):
    nodes = [yv[k : k + 1, :] for k in range(k0, k1)]
    size = 1
    while len(nodes) > 1:
        nxt = []
        for j in range(len(nodes) // 2):
            m = k0 + 2 * size * j + size - 1
            mask = xb > t[m : m + 1, :]
            nxt.append(jnp.where(mask, nodes[2 * j + 1], nodes[2 * j]))
        nodes = nxt
        size *= 2
    return nodes[0]


def _codec_block_kernel(x_ref, t_ref, y_ref, o_ref):
    xb = x_ref[...]
    t = t_ref[...]
    yv = y_ref[...]
    lo = _subtree(xb, t, yv, 0, _K // 2)
    hi = _subtree(xb, t, yv, _K // 2, _K)
    o_ref[...] = jnp.where(xb > t[_K // 2 - 1 : _K // 2, :], hi, lo)


def _tc_codec(x2, t_pad, y_t, tn, th, row0):
    n, h = x2.shape
    grid = (h // th, (n - row0) // tn)
    off = row0 // tn
    return pl.pallas_call(
        _codec_block_kernel,
        grid=grid,
        in_specs=[
            pl.BlockSpec((tn, th), lambda j, i: (i + off, j)),
            pl.BlockSpec((_K, th), lambda j, i: (0, j)),
            pl.BlockSpec((_K, th), lambda j, i: (0, j)),
        ],
        out_specs=pl.BlockSpec((tn, th), lambda j, i: (i + off, j)),
        out_shape=jax.ShapeDtypeStruct((n, h), jnp.float32),
        compiler_params=pltpu.CompilerParams(
            dimension_semantics=("arbitrary", "arbitrary"),
        ),
    )(x2, t_pad, y_t)


def _sc_codec(x_sc, t_cm, y_cm):
    n_sc, h = x_sc.shape
    mesh = plsc.VectorSubcoreMesh(core_axis_name="c", subcore_axis_name="s")
    grid = (n_sc // _SC_TT, h // _SC_CH)

    cp = pltpu.CompilerParams()
    if "needs_layout_passes" in pltpu.CompilerParams.__dataclass_fields__:
        cp = dataclasses.replace(cp, needs_layout_passes=False)

    @functools.partial(
        pl.kernel,
        out_type=jax.ShapeDtypeStruct((n_sc, h), jnp.float32),
        mesh=mesh,
        compiler_params=cp,
    )
    def run(x_hbm, t_hbm, y_hbm, o_hbm):
        def body(x_v, t_v, y_v, o_v):
            ci = jax.lax.iota(jnp.int32, 16)

            @pl.loop(0, _SC_TT)
            def _(ti):
                for g in range(_SC_CH // 16):
                    xv = x_v.at[ti, pl.ds(16 * g, 16)][...]
                    cg = ci + (16 * g)
                    idx = jnp.zeros((16,), jnp.int32)
                    for s in (16, 8, 4, 2, 1):
                        tv = plsc.load_gather(t_v, [cg, idx + (s - 1)])
                        idx = idx + jnp.where(xv > tv, jnp.int32(s), jnp.int32(0))
                    ov = plsc.load_gather(y_v, [cg, idx])
                    o_v.at[ti, pl.ds(16 * g, 16)][...] = ov

        pltpu.emit_pipeline(
            body,
            grid=grid,
            in_specs=[
                pl.BlockSpec((_SC_TT, _SC_CH), lambda i, j: (i, j)),
                pl.BlockSpec((_SC_CH, _K_PAD), lambda i, j: (j, 0)),
                pl.BlockSpec((_SC_CH, _K_PAD), lambda i, j: (j, 0)),
            ],
            out_specs=[pl.BlockSpec((_SC_TT, _SC_CH), lambda i, j: (i, j))],
            core_axis_name=("c", "s"),
            dimension_semantics=(pltpu.PARALLEL, pltpu.PARALLEL),
        )(x_hbm, t_hbm, y_hbm, o_hbm)

    return run(x_sc, t_cm, y_cm)


@functools.partial(jax.jit, static_argnames=("tn", "th", "n_sc"))
def _codec(x2, t_pad, y_t, t_cm, y_cm, tn, th, n_sc):
    n, h = x2.shape
    out_sc = _sc_codec(x2[:n_sc], t_cm, y_cm)
    if n_sc == n:
        return out_sc
    out = _tc_codec(x2, t_pad, y_t, tn, th, row0=n_sc)
    return jax.lax.dynamic_update_slice(out, out_sc, (0, 0))


def kernel(x, thresholds, y):
    shape = x.shape
    h = shape[-1]
    x2 = x.reshape(-1, h).astype(jnp.float32)
    t_t = thresholds.T.astype(jnp.float32)
    t_pad = jnp.concatenate([t_t, t_t[-1:, :]], axis=0)
    y_t = y.T.astype(jnp.float32)
    tf = thresholds.astype(jnp.float32)
    t_cm = jnp.concatenate([tf] + [tf[:, -1:]] * (_K_PAD - _K + 1), 1)
    yf = y.astype(jnp.float32)
    y_cm = jnp.concatenate([yf, yf[:, -1:]], 1)
    out = _codec(x2, t_pad, y_t, t_cm, y_cm, tn=2048, th=128, n_sc=2048)
    return out.reshape(shape)


# speedup vs baseline: 1.6859x; 1.6859x over previous
import dataclasses
import functools

import jax
import jax.numpy as jnp
from jax.experimental import pallas as pl
from jax.experimental.pallas import tpu as pltpu
from jax.experimental.pallas import tpu_sc as plsc

_K = 32
_K_PAD = 33
_SC_CH = 128
_SC_TT = 64


def _subtree(xb, t, yv, k0, k1):
    nodes = [yv[k : k + 1, :] for k in range(k0, k1)]
    size = 1
    while len(nodes) > 1:
        nxt = []
        for j in range(len(nodes) // 2):
            m = k0 + 2 * size * j + size - 1
            mask = xb > t[m : m + 1, :]
            nxt.append(jnp.where(mask, nodes[2 * j + 1], nodes[2 * j]))
        nodes = nxt
        size *= 2
    return nodes[0]


def _codec_block_kernel(x_ref, t_ref, y_ref, o_ref):
    xb = x_ref[...]
    t = t_ref[...]
    yv = y_ref[...]
    lo = _subtree(xb, t, yv, 0, _K // 2)
    hi = _subtree(xb, t, yv, _K // 2, _K)
    o_ref[...] = jnp.where(xb > t[_K // 2 - 1 : _K // 2, :], hi, lo)


def _tc_codec(x2, t_pad, y_t, tn, th, row0):
    n, h = x2.shape
    grid = (h // th, (n - row0) // tn)
    off = row0 // tn
    return pl.pallas_call(
        _codec_block_kernel,
        grid=grid,
        in_specs=[
            pl.BlockSpec((tn, th), lambda j, i: (i + off, j)),
            pl.BlockSpec((_K, th), lambda j, i: (0, j)),
            pl.BlockSpec((_K, th), lambda j, i: (0, j)),
        ],
        out_specs=pl.BlockSpec((tn, th), lambda j, i: (i + off, j)),
        out_shape=jax.ShapeDtypeStruct((n, h), jnp.float32),
        compiler_params=pltpu.CompilerParams(
            dimension_semantics=("arbitrary", "arbitrary"),
        ),
    )(x2, t_pad, y_t)


def _sc_codec(x_sc, t_cm, y_cm):
    n_sc, h = x_sc.shape
    mesh = plsc.VectorSubcoreMesh(core_axis_name="c", subcore_axis_name="s")
    grid = (n_sc // _SC_TT, h // _SC_CH)

    cp = pltpu.CompilerParams()
    if "needs_layout_passes" in pltpu.CompilerParams.__dataclass_fields__:
        cp = dataclasses.replace(cp, needs_layout_passes=False)

    @functools.partial(
        pl.kernel,
        out_type=jax.ShapeDtypeStruct((n_sc, h), jnp.float32),
        mesh=mesh,
        compiler_params=cp,
    )
    def run(x_hbm, t_hbm, y_hbm, o_hbm):
        def body(x_v, t_v, y_v, o_v):
            ci = jax.lax.iota(jnp.int32, 16)
            ngroup = _SC_CH // 16

            @pl.loop(0, _SC_TT)
            def _(ti):
                xs = [x_v.at[ti, pl.ds(16 * g, 16)][...] for g in range(ngroup)]
                cgs = [ci + (16 * g) for g in range(ngroup)]
                idxs = [jnp.zeros((16,), jnp.int32)] * ngroup
                for s in (16, 8, 4, 2, 1):
                    tvs = [
                        plsc.load_gather(t_v, [cgs[g], idxs[g] + (s - 1)])
                        for g in range(ngroup)
                    ]
                    idxs = [
                        idxs[g]
                        + jnp.where(xs[g] > tvs[g], jnp.int32(s), jnp.int32(0))
                        for g in range(ngroup)
                    ]
                for g in range(ngroup):
                    ov = plsc.load_gather(y_v, [cgs[g], idxs[g]])
                    o_v.at[ti, pl.ds(16 * g, 16)][...] = ov

        pltpu.emit_pipeline(
            body,
            grid=grid,
            in_specs=[
                pl.BlockSpec((_SC_TT, _SC_CH), lambda i, j: (i, j)),
                pl.BlockSpec((_SC_CH, _K_PAD), lambda i, j: (j, 0)),
                pl.BlockSpec((_SC_CH, _K_PAD), lambda i, j: (j, 0)),
            ],
            out_specs=[pl.BlockSpec((_SC_TT, _SC_CH), lambda i, j: (i, j))],
            core_axis_name=("c", "s"),
            dimension_semantics=(pltpu.PARALLEL, pltpu.PARALLEL),
        )(x_hbm, t_hbm, y_hbm, o_hbm)

    return run(x_sc, t_cm, y_cm)


@functools.partial(jax.jit, static_argnames=("tn", "th", "n_sc"))
def _codec(x2, t_pad, y_t, t_cm, y_cm, tn, th, n_sc):
    n, h = x2.shape
    out_sc = _sc_codec(x2[:n_sc], t_cm, y_cm)
    if n_sc == n:
        return out_sc
    out = _tc_codec(x2, t_pad, y_t, tn, th, row0=n_sc)
    return jax.lax.dynamic_update_slice(out, out_sc, (0, 0))


def kernel(x, thresholds, y):
    shape = x.shape
    h = shape[-1]
    x2 = x.reshape(-1, h).astype(jnp.float32)
    t_t = thresholds.T.astype(jnp.float32)
    t_pad = jnp.concatenate([t_t, t_t[-1:, :]], axis=0)
    y_t = y.T.astype(jnp.float32)
    tf = thresholds.astype(jnp.float32)
    t_cm = jnp.concatenate([tf] + [tf[:, -1:]] * (_K_PAD - _K + 1), 1)
    yf = y.astype(jnp.float32)
    y_cm = jnp.concatenate([yf, yf[:, -1:]], 1)
    out = _codec(x2, t_pad, y_t, t_cm, y_cm, tn=2048, th=128, n_sc=2048)
    return out.reshape(shape)
